# baseline (speedup 1.0000x reference)
import functools

import jax
import jax.numpy as jnp
from jax.experimental import pallas as pl
from jax.experimental.pallas import tpu as pltpu


def _node_layer1_kernel(x0_ref, w_ref, a_ref, o_ref, abf_ref, m0_ref):
    i = pl.program_id(1)

    @pl.when(i == 0)
    def _():
        m0 = jnp.dot(x0_ref[...], w_ref[...], preferred_element_type=jnp.float32)
        m0_ref[...] = m0.astype(jnp.bfloat16)

    a_bf = a_ref[...].astype(jnp.bfloat16)
    abf_ref[...] = a_bf
    h = jnp.dot(a_bf, m0_ref[...], preferred_element_type=jnp.float32)
    o_ref[...] = jnp.maximum(h, 0.0)


def _node_layer_kernel(x0_ref, w_ref, a_ref, o_ref, m0_ref):
    i = pl.program_id(1)

    @pl.when(i == 0)
    def _():
        m0 = jnp.dot(x0_ref[...], w_ref[...], preferred_element_type=jnp.float32)
        m0_ref[...] = m0.astype(jnp.bfloat16)

    h = jnp.dot(a_ref[...], m0_ref[...], preferred_element_type=jnp.float32)
    o_ref[...] = jnp.maximum(h, 0.0)


def _face_kernel(x1_ref, w_ref, inc_ref, o_ref, m1_ref):
    i = pl.program_id(1)

    @pl.when(i == 0)
    def _():
        m1 = jnp.dot(x1_ref[...].astype(jnp.bfloat16), w_ref[...].astype(jnp.bfloat16),
                     preferred_element_type=jnp.float32)
        m1_ref[...] = m1.astype(jnp.bfloat16)

    h = jnp.dot(inc_ref[...].astype(jnp.bfloat16), m1_ref[...],
                preferred_element_type=jnp.float32)
    o_ref[...] = jnp.maximum(h, 0.0)


def kernel(x_0, x_1, adjacency_0, incidence_2_t, w0_stack, w12_stack):
    n_nodes, c0 = x_0.shape
    n_edges, c1 = x_1.shape
    n_faces = incidence_2_t.shape[0]
    n_layers = w0_stack.shape[0]
    c2 = w12_stack.shape[2]

    tm = 512
    t_per_core = n_nodes // tm // 2

    x0_cur, a_bf16 = pl.pallas_call(
        _node_layer1_kernel,
        grid=(2, t_per_core),
        out_shape=(
            jax.ShapeDtypeStruct((n_nodes, c0), x_0.dtype),
            jax.ShapeDtypeStruct((n_nodes, n_nodes), jnp.bfloat16),
        ),
        in_specs=[
            pl.BlockSpec((n_nodes, c0), lambda c, i: (0, 0)),
            pl.BlockSpec((c0, c0), lambda c, i: (0, 0)),
            pl.BlockSpec((tm, n_nodes), lambda c, i: (c * (n_nodes // tm // 2) + i, 0)),
        ],
        out_specs=(
            pl.BlockSpec((tm, c0), lambda c, i: (c * (n_nodes // tm // 2) + i, 0)),
            pl.BlockSpec((tm, n_nodes), lambda c, i: (c * (n_nodes // tm // 2) + i, 0)),
        ),
        scratch_shapes=[pltpu.VMEM((n_nodes, c0), jnp.bfloat16)],
        compiler_params=pltpu.CompilerParams(
            dimension_semantics=("parallel", "arbitrary")),
    )(x_0, w0_stack[0], adjacency_0)

    for l in range(1, n_layers):
        x0_cur = pl.pallas_call(
            _node_layer_kernel,
            grid=(2, t_per_core),
            out_shape=jax.ShapeDtypeStruct((n_nodes, c0), x_0.dtype),
            in_specs=[
                pl.BlockSpec((n_nodes, c0), lambda c, i: (0, 0)),
                pl.BlockSpec((c0, c0), lambda c, i: (0, 0)),
                pl.BlockSpec((tm, n_nodes), lambda c, i: (c * (n_nodes // tm // 2) + i, 0)),
            ],
            out_specs=pl.BlockSpec((tm, c0), lambda c, i: (c * (n_nodes // tm // 2) + i, 0)),
            scratch_shapes=[pltpu.VMEM((n_nodes, c0), jnp.bfloat16)],
            compiler_params=pltpu.CompilerParams(
                dimension_semantics=("parallel", "arbitrary")),
        )(x0_cur, w0_stack[l], a_bf16)

    tf = 512
    tf_per_core = n_faces // tf // 2
    x2 = pl.pallas_call(
        _face_kernel,
        grid=(2, tf_per_core),
        out_shape=jax.ShapeDtypeStruct((n_faces, c2), x_1.dtype),
        in_specs=[
            pl.BlockSpec((n_edges, c1), lambda c, i: (0, 0)),
            pl.BlockSpec((c1, c2), lambda c, i: (0, 0)),
            pl.BlockSpec((tf, n_edges), lambda c, i: (c * (n_faces // tf // 2) + i, 0)),
        ],
        out_specs=pl.BlockSpec((tf, c2), lambda c, i: (c * (n_faces // tf // 2) + i, 0)),
        scratch_shapes=[pltpu.VMEM((n_edges, c2), jnp.bfloat16)],
        compiler_params=pltpu.CompilerParams(
            dimension_semantics=("parallel", "arbitrary")),
    )(x_1, w12_stack[n_layers - 1], incidence_2_t)

    return x0_cur, x_1, x2


# speedup vs baseline: 1.2208x; 1.2208x over previous
import functools

import jax
import jax.numpy as jnp
from jax.experimental import pallas as pl
from jax.experimental.pallas import tpu as pltpu


def _node_kernel(x0_ref, w0_ref, a_ref, o_ref, abf_ref, m0_ref, *, tm, n_tiles):
    l = pl.program_id(0)
    i = pl.program_id(1)

    @pl.when(jnp.logical_and(i == 0, l == 0))
    def _():
        m0 = jnp.dot(x0_ref[...], w0_ref[0], preferred_element_type=jnp.float32)
        m0_ref[...] = m0.astype(jnp.bfloat16)

    @pl.when(jnp.logical_and(i == 0, l > 0))
    def _():
        m0 = jnp.dot(o_ref[...], w0_ref[0], preferred_element_type=jnp.float32)
        m0_ref[...] = m0.astype(jnp.bfloat16)

    row = pl.multiple_of(i * tm, tm)

    @pl.when(l == 0)
    def _():
        a_bf = a_ref[...].astype(jnp.bfloat16)
        abf_ref[pl.ds(row, tm), :] = a_bf
        h = jnp.dot(a_bf, m0_ref[...], preferred_element_type=jnp.float32)
        o_ref[pl.ds(row, tm), :] = jnp.maximum(h, 0.0)

    @pl.when(l > 0)
    def _():
        h = jnp.dot(abf_ref[pl.ds(row, tm), :], m0_ref[...],
                    preferred_element_type=jnp.float32)
        o_ref[pl.ds(row, tm), :] = jnp.maximum(h, 0.0)


def _face_kernel(x1_ref, w12_ref, inc_ref, o2_ref, m1_ref):
    i = pl.program_id(1)

    @pl.when(i == 0)
    def _():
        m1 = jnp.dot(x1_ref[...].astype(jnp.bfloat16),
                     w12_ref[...].astype(jnp.bfloat16),
                     preferred_element_type=jnp.float32)
        m1_ref[...] = m1.astype(jnp.bfloat16)

    h = jnp.dot(inc_ref[...].astype(jnp.bfloat16), m1_ref[...],
                preferred_element_type=jnp.float32)
    o2_ref[...] = jnp.maximum(h, 0.0)


def kernel(x_0, x_1, adjacency_0, incidence_2_t, w0_stack, w12_stack):
    n_nodes, c0 = x_0.shape
    n_edges, c1 = x_1.shape
    n_faces = incidence_2_t.shape[0]
    n_layers = w0_stack.shape[0]
    c2 = w12_stack.shape[2]

    tm = 256
    n_tiles = n_nodes // tm

    x0_out = pl.pallas_call(
        functools.partial(_node_kernel, tm=tm, n_tiles=n_tiles),
        grid=(n_layers, n_tiles),
        out_shape=jax.ShapeDtypeStruct((n_nodes, c0), x_0.dtype),
        in_specs=[
            pl.BlockSpec((n_nodes, c0), lambda l, i: (0, 0)),
            pl.BlockSpec((1, c0, c0), lambda l, i: (l, 0, 0)),
            pl.BlockSpec((tm, n_nodes),
                         lambda l, i: (jnp.where(l == 0, i, n_tiles - 1), 0)),
        ],
        out_specs=pl.BlockSpec((n_nodes, c0), lambda l, i: (0, 0)),
        scratch_shapes=[
            pltpu.VMEM((n_nodes, n_nodes), jnp.bfloat16),
            pltpu.VMEM((n_nodes, c0), jnp.bfloat16),
        ],
        compiler_params=pltpu.CompilerParams(
            dimension_semantics=("arbitrary", "arbitrary")),
    )(x_0, w0_stack, adjacency_0)

    tf = 256
    tf_per_core = n_faces // tf // 2
    x2 = pl.pallas_call(
        _face_kernel,
        grid=(2, tf_per_core),
        out_shape=jax.ShapeDtypeStruct((n_faces, c2), x_1.dtype),
        in_specs=[
            pl.BlockSpec((n_edges, c1), lambda c, i: (0, 0)),
            pl.BlockSpec((c1, c2), lambda c, i: (0, 0)),
            pl.BlockSpec((tf, n_edges), lambda c, i: (c * (n_faces // tf // 2) + i, 0)),
        ],
        out_specs=pl.BlockSpec((tf, c2), lambda c, i: (c * (n_faces // tf // 2) + i, 0)),
        scratch_shapes=[pltpu.VMEM((n_edges, c2), jnp.bfloat16)],
        compiler_params=pltpu.CompilerParams(
            dimension_semantics=("parallel", "arbitrary")),
    )(x_1, w12_stack[n_layers - 1], incidence_2_t)

    return x0_out, x_1, x2


# speedup vs baseline: 1.2360x; 1.0124x over previous
import functools

import jax
import jax.numpy as jnp
from jax.experimental import pallas as pl
from jax.experimental.pallas import tpu as pltpu


def _node_kernel(x0_ref, w0_ref, a_ref, oT_ref, aT_ref, m0T_ref, *, tm, n_tiles):
    l = pl.program_id(0)
    i = pl.program_id(1)

    @pl.when(jnp.logical_and(i == 0, l == 0))
    def _():
        m0T = jax.lax.dot_general(
            w0_ref[0], x0_ref[...], (((0,), (1,)), ((), ())),
            preferred_element_type=jnp.float32)
        m0T_ref[...] = m0T.astype(jnp.bfloat16)

    @pl.when(jnp.logical_and(i == 0, l > 0))
    def _():
        m0T = jax.lax.dot_general(
            w0_ref[0], oT_ref[...], (((0,), (0,)), ((), ())),
            preferred_element_type=jnp.float32)
        m0T_ref[...] = m0T.astype(jnp.bfloat16)

    col = pl.multiple_of(i * tm, tm)

    @pl.when(l == 0)
    def _():
        a_bf = a_ref[...].astype(jnp.bfloat16)
        aT = jnp.swapaxes(a_bf, 0, 1)
        aT_ref[:, pl.ds(col, tm)] = aT
        h = jax.lax.dot_general(m0T_ref[...], aT, (((1,), (0,)), ((), ())),
                                preferred_element_type=jnp.float32)
        oT_ref[:, pl.ds(col, tm)] = jnp.maximum(h, 0.0)

    @pl.when(l > 0)
    def _():
        h = jax.lax.dot_general(m0T_ref[...], aT_ref[:, pl.ds(col, tm)],
                                (((1,), (0,)), ((), ())),
                                preferred_element_type=jnp.float32)
        oT_ref[:, pl.ds(col, tm)] = jnp.maximum(h, 0.0)


def _face_kernel(x1_ref, w12_ref, inc_ref, o2_ref, m1_ref):
    i = pl.program_id(1)

    @pl.when(i == 0)
    def _():
        m1 = jnp.dot(x1_ref[...].astype(jnp.bfloat16),
                     w12_ref[...].astype(jnp.bfloat16),
                     preferred_element_type=jnp.float32)
        m1_ref[...] = m1.astype(jnp.bfloat16)

    h = jnp.dot(inc_ref[...].astype(jnp.bfloat16), m1_ref[...],
                preferred_element_type=jnp.float32)
    o2_ref[...] = jnp.maximum(h, 0.0)


def kernel(x_0, x_1, adjacency_0, incidence_2_t, w0_stack, w12_stack):
    n_nodes, c0 = x_0.shape
    n_edges, c1 = x_1.shape
    n_faces = incidence_2_t.shape[0]
    n_layers = w0_stack.shape[0]
    c2 = w12_stack.shape[2]

    tm = 256
    n_tiles = n_nodes // tm

    x0T_out = pl.pallas_call(
        functools.partial(_node_kernel, tm=tm, n_tiles=n_tiles),
        grid=(n_layers, n_tiles),
        out_shape=jax.ShapeDtypeStruct((c0, n_nodes), x_0.dtype),
        in_specs=[
            pl.BlockSpec((n_nodes, c0), lambda l, i: (0, 0)),
            pl.BlockSpec((1, c0, c0), lambda l, i: (l, 0, 0)),
            pl.BlockSpec((tm, n_nodes),
                         lambda l, i: (jnp.where(l == 0, i, n_tiles - 1), 0)),
        ],
        out_specs=pl.BlockSpec((c0, n_nodes), lambda l, i: (0, 0)),
        scratch_shapes=[
            pltpu.VMEM((n_nodes, n_nodes), jnp.bfloat16),
            pltpu.VMEM((c0, n_nodes), jnp.bfloat16),
        ],
        compiler_params=pltpu.CompilerParams(
            dimension_semantics=("arbitrary", "arbitrary")),
    )(x_0, w0_stack, adjacency_0)

    x0_out = jnp.transpose(x0T_out)

    tf = 256
    tf_per_core = n_faces // tf // 2
    x2 = pl.pallas_call(
        _face_kernel,
        grid=(2, tf_per_core),
        out_shape=jax.ShapeDtypeStruct((n_faces, c2), x_1.dtype),
        in_specs=[
            pl.BlockSpec((n_edges, c1), lambda c, i: (0, 0)),
            pl.BlockSpec((c1, c2), lambda c, i: (0, 0)),
            pl.BlockSpec((tf, n_edges), lambda c, i: (c * (n_faces // tf // 2) + i, 0)),
        ],
        out_specs=pl.BlockSpec((tf, c2), lambda c, i: (c * (n_faces // tf // 2) + i, 0)),
        scratch_shapes=[pltpu.VMEM((n_edges, c2), jnp.bfloat16)],
        compiler_params=pltpu.CompilerParams(
            dimension_semantics=("parallel", "arbitrary")),
    )(x_1, w12_stack[n_layers - 1], incidence_2_t)

    return x0_out, x_1, x2


# speedup vs baseline: 1.4775x; 1.1954x over previous
import functools

import jax
import jax.numpy as jnp
from jax.experimental import pallas as pl
from jax.experimental.pallas import tpu as pltpu

_TM0 = 512
_TC = 2048


def _node_kernel(x0_ref, w0_ref, a_ref, oT_ref, aT_ref, m0T_ref, *, nt0, nc, n_layers):
    s = pl.program_id(0)
    n = x0_ref.shape[0]

    @pl.when(s == 0)
    def _():
        m0T = jax.lax.dot_general(
            w0_ref[0], x0_ref[...], (((0,), (1,)), ((), ())),
            preferred_element_type=jnp.float32)
        m0T_ref[...] = m0T.astype(jnp.bfloat16)

    @pl.when(s < nt0)
    def _():
        col = pl.multiple_of(s * _TM0, _TM0)
        a_bf = a_ref[...].astype(jnp.bfloat16)
        aT = jnp.swapaxes(a_bf, 0, 1)
        aT_ref[:, pl.ds(col, _TM0)] = aT
        h = jax.lax.dot_general(m0T_ref[...], aT, (((1,), (0,)), ((), ())),
                                preferred_element_type=jnp.float32)
        oT_ref[:, pl.ds(col, _TM0)] = jnp.maximum(h, 0.0)

    @pl.when(jnp.logical_and(s >= nt0, (s - nt0) % nc == 0))
    def _():
        m0T = jax.lax.dot_general(
            w0_ref[0], oT_ref[...], (((0,), (0,)), ((), ())),
            preferred_element_type=jnp.float32)
        m0T_ref[...] = m0T.astype(jnp.bfloat16)

    @pl.when(s >= nt0)
    def _():
        col = pl.multiple_of(((s - nt0) % nc) * _TC, _TC)
        h = jax.lax.dot_general(m0T_ref[...], aT_ref[:, pl.ds(col, _TC)],
                                (((1,), (0,)), ((), ())),
                                preferred_element_type=jnp.float32)
        oT_ref[:, pl.ds(col, _TC)] = jnp.maximum(h, 0.0)


def _face_kernel(x1_ref, w12_ref, inc_ref, o2_ref, m1_ref):
    i = pl.program_id(1)

    @pl.when(i == 0)
    def _():
        m1 = jnp.dot(x1_ref[...].astype(jnp.bfloat16),
                     w12_ref[...].astype(jnp.bfloat16),
                     preferred_element_type=jnp.float32)
        m1_ref[...] = m1.astype(jnp.bfloat16)

    h = jnp.dot(inc_ref[...].astype(jnp.bfloat16), m1_ref[...],
                preferred_element_type=jnp.float32)
    o2_ref[...] = jnp.maximum(h, 0.0)


def kernel(x_0, x_1, adjacency_0, incidence_2_t, w0_stack, w12_stack):
    n_nodes, c0 = x_0.shape
    n_edges, c1 = x_1.shape
    n_faces = incidence_2_t.shape[0]
    n_layers = w0_stack.shape[0]
    c2 = w12_stack.shape[2]

    nt0 = n_nodes // _TM0
    nc = n_nodes // _TC
    n_steps = nt0 + (n_layers - 1) * nc

    x0T_out = pl.pallas_call(
        functools.partial(_node_kernel, nt0=nt0, nc=nc, n_layers=n_layers),
        grid=(n_steps,),
        out_shape=jax.ShapeDtypeStruct((c0, n_nodes), x_0.dtype),
        in_specs=[
            pl.BlockSpec((n_nodes, c0), lambda s: (0, 0)),
            pl.BlockSpec((1, c0, c0),
                         lambda s: (jnp.where(s < nt0, 0, (s - nt0) // nc + 1), 0, 0)),
            pl.BlockSpec((_TM0, n_nodes),
                         lambda s: (jnp.minimum(s, nt0 - 1), 0)),
        ],
        out_specs=pl.BlockSpec((c0, n_nodes), lambda s: (0, 0)),
        scratch_shapes=[
            pltpu.VMEM((n_nodes, n_nodes), jnp.bfloat16),
            pltpu.VMEM((c0, n_nodes), jnp.bfloat16),
        ],
        compiler_params=pltpu.CompilerParams(
            dimension_semantics=("arbitrary",)),
    )(x_0, w0_stack, adjacency_0)

    x0_out = jnp.transpose(x0T_out)

    tf = 256
    tf_per_core = n_faces // tf // 2
    x2 = pl.pallas_call(
        _face_kernel,
        grid=(2, tf_per_core),
        out_shape=jax.ShapeDtypeStruct((n_faces, c2), x_1.dtype),
        in_specs=[
            pl.BlockSpec((n_edges, c1), lambda c, i: (0, 0)),
            pl.BlockSpec((c1, c2), lambda c, i: (0, 0)),
            pl.BlockSpec((tf, n_edges), lambda c, i: (c * (n_faces // tf // 2) + i, 0)),
        ],
        out_specs=pl.BlockSpec((tf, c2), lambda c, i: (c * (n_faces // tf // 2) + i, 0)),
        scratch_shapes=[pltpu.VMEM((n_edges, c2), jnp.bfloat16)],
        compiler_params=pltpu.CompilerParams(
            dimension_semantics=("parallel", "arbitrary")),
    )(x_1, w12_stack[n_layers - 1], incidence_2_t)

    return x0_out, x_1, x2
